# baseline (speedup 1.0000x reference)
import functools

import jax
import jax.numpy as jnp
from jax import lax
from jax.experimental import pallas as pl

N_DIM = 256
E_DIM = 128
A_DIM = 64
AXIS = 4
NLOC = 10000
NALL = 12000
NNEI = 16
NEDGE = 160000
NANGLE = 160000
DYN_E = NNEI / 10.0
DYN_A = 16 / 10.0

EBLK = 1280
NBLK = 1000


def _silu(x):
    return x * (1.0 / (1.0 + jnp.exp(-x)))


def _full_spec(shape):
    return pl.BlockSpec(shape, lambda i: tuple(0 for _ in shape))


def _row_spec(blk, shape):
    return pl.BlockSpec((blk,) + shape[1:], lambda i: (i,) + tuple(0 for _ in shape[1:]))


def _edge_body(ni, nn, ee, sw, h2, A1, A2, A3, bne, B1, B2, B3, bes, eres,
               neu, epart, w3):
    x_ni = ni[...]
    x_nn = nn[...]
    x_ee = ee[...]
    s = sw[...]
    dot = functools.partial(jnp.dot, preferred_element_type=jnp.float32)
    pre_u = dot(x_ni, A1[...]) + dot(x_nn, A2[...]) + dot(x_ee, A3[...]) + bne[...]
    neu[...] = _silu(pre_u) * s
    pre_e = dot(x_ni, B1[...]) + dot(x_nn, B2[...]) + dot(x_ee, B3[...]) + bes[...]
    epart[...] = x_ee + eres[...] * _silu(pre_e)
    w3[...] = h2[...] * s


def _tc_edge_mlp(node_i, nei, edge_ebd, sw2, h2, W_ne, b_ne, W_es, b_es, e_res0):
    grid = (NEDGE // EBLK,)
    out_shapes = (
        jax.ShapeDtypeStruct((NEDGE, N_DIM), jnp.float32),
        jax.ShapeDtypeStruct((NEDGE, E_DIM), jnp.float32),
        jax.ShapeDtypeStruct((NEDGE, 3), jnp.float32),
    )
    A1, A2, A3 = W_ne[:N_DIM], W_ne[N_DIM:2 * N_DIM], W_ne[2 * N_DIM:]
    B1, B2, B3 = W_es[:N_DIM], W_es[N_DIM:2 * N_DIM], W_es[2 * N_DIM:]
    return pl.pallas_call(
        _edge_body,
        grid=grid,
        in_specs=[
            _row_spec(EBLK, (NEDGE, N_DIM)),
            _row_spec(EBLK, (NEDGE, N_DIM)),
            _row_spec(EBLK, (NEDGE, E_DIM)),
            _row_spec(EBLK, (NEDGE, 1)),
            _row_spec(EBLK, (NEDGE, 3)),
            _full_spec((N_DIM, N_DIM)),
            _full_spec((N_DIM, N_DIM)),
            _full_spec((E_DIM, N_DIM)),
            _full_spec((1, N_DIM)),
            _full_spec((N_DIM, E_DIM)),
            _full_spec((N_DIM, E_DIM)),
            _full_spec((E_DIM, E_DIM)),
            _full_spec((1, E_DIM)),
            _full_spec((1, E_DIM)),
        ],
        out_specs=[
            _row_spec(EBLK, (NEDGE, N_DIM)),
            _row_spec(EBLK, (NEDGE, E_DIM)),
            _row_spec(EBLK, (NEDGE, 3)),
        ],
        out_shape=out_shapes,
    )(node_i, nei, edge_ebd, sw2, h2, A1, A2, A3, b_ne[None, :],
      B1, B2, B3, b_es[None, :], e_res0[None, :])


def _angle_body(ab, na, ik, ij, asw, C1, C2, C3, C4, bea, D1, D2, D3, D4, bas,
                ares, eaw, aupd):
    x_ab = ab[...]
    x_na = na[...]
    x_ik = ik[...]
    x_ij = ij[...]
    dot = functools.partial(jnp.dot, preferred_element_type=jnp.float32)
    pre_e = (dot(x_ab, C1[...]) + dot(x_na, C2[...]) + dot(x_ik, C3[...])
             + dot(x_ij, C4[...]) + bea[...])
    eaw[...] = _silu(pre_e) * asw[...]
    pre_a = (dot(x_ab, D1[...]) + dot(x_na, D2[...]) + dot(x_ik, D3[...])
             + dot(x_ij, D4[...]) + bas[...])
    aupd[...] = x_ab + ares[...] * _silu(pre_a)


def _tc_angle_mlp(angle_ebd, node_a, edge_ik, edge_ij, a_sw2, W_ea1, b_ea1,
                  W_as, b_as, a_res0):
    grid = (NANGLE // EBLK,)
    C1 = W_ea1[:A_DIM]
    C2 = W_ea1[A_DIM:A_DIM + N_DIM]
    C3 = W_ea1[A_DIM + N_DIM:A_DIM + N_DIM + E_DIM]
    C4 = W_ea1[A_DIM + N_DIM + E_DIM:]
    D1 = W_as[:A_DIM]
    D2 = W_as[A_DIM:A_DIM + N_DIM]
    D3 = W_as[A_DIM + N_DIM:A_DIM + N_DIM + E_DIM]
    D4 = W_as[A_DIM + N_DIM + E_DIM:]
    return pl.pallas_call(
        _angle_body,
        grid=grid,
        in_specs=[
            _row_spec(EBLK, (NANGLE, A_DIM)),
            _row_spec(EBLK, (NANGLE, N_DIM)),
            _row_spec(EBLK, (NANGLE, E_DIM)),
            _row_spec(EBLK, (NANGLE, E_DIM)),
            _row_spec(EBLK, (NANGLE, 1)),
            _full_spec((A_DIM, E_DIM)),
            _full_spec((N_DIM, E_DIM)),
            _full_spec((E_DIM, E_DIM)),
            _full_spec((E_DIM, E_DIM)),
            _full_spec((1, E_DIM)),
            _full_spec((A_DIM, A_DIM)),
            _full_spec((N_DIM, A_DIM)),
            _full_spec((E_DIM, A_DIM)),
            _full_spec((E_DIM, A_DIM)),
            _full_spec((1, A_DIM)),
            _full_spec((1, A_DIM)),
        ],
        out_specs=[
            _row_spec(EBLK, (NANGLE, E_DIM)),
            _row_spec(EBLK, (NANGLE, A_DIM)),
        ],
        out_shape=(
            jax.ShapeDtypeStruct((NANGLE, E_DIM), jnp.float32),
            jax.ShapeDtypeStruct((NANGLE, A_DIM), jnp.float32),
        ),
    )(angle_ebd, node_a, edge_ik, edge_ij, a_sw2, C1, C2, C3, C4, b_ea1[None, :],
      D1, D2, D3, D4, b_as[None, :], a_res0[None, :])


def _node_body(xr, her, hnr, msgr, Wns, bns, Wsym, bsym, nr0, nr1, nr2, out):
    x = xr[...]
    dot = functools.partial(jnp.dot, preferred_element_type=jnp.float32)
    node_self = _silu(dot(x, Wns[...]) + bns[...])
    F = 1.0 / (DYN_E * 3.0)
    pre = jnp.zeros_like(x) + bsym[...]
    he = [her[c] for c in range(3)]
    hn = [hnr[c] for c in range(3)]
    for a in range(AXIS):
        ge_a = (he[0][:, a:a + 1] * he[0] + he[1][:, a:a + 1] * he[1]
                + he[2][:, a:a + 1] * he[2]) * F
        pre += dot(ge_a, Wsym[a * E_DIM:(a + 1) * E_DIM, :])
        gn_a = (hn[0][:, a:a + 1] * hn[0] + hn[1][:, a:a + 1] * hn[1]
                + hn[2][:, a:a + 1] * hn[2]) * F
        base = AXIS * E_DIM + a * N_DIM
        pre += dot(gn_a, Wsym[base:base + N_DIM, :])
    node_sym = _silu(pre)
    out[...] = (x + nr0[...] * node_self + nr1[...] * node_sym
                + nr2[...] * (msgr[...] * (1.0 / DYN_E)))


def _tc_node(node_ebd, h2g2_e, h2g2_n, msg, W_ns, b_ns, W_sym, b_sym,
             n_res0, n_res1, n_res2):
    grid = (NLOC // NBLK,)
    return pl.pallas_call(
        _node_body,
        grid=grid,
        in_specs=[
            _row_spec(NBLK, (NLOC, N_DIM)),
            pl.BlockSpec((3, NBLK, E_DIM), lambda i: (0, i, 0)),
            pl.BlockSpec((3, NBLK, N_DIM), lambda i: (0, i, 0)),
            _row_spec(NBLK, (NLOC, N_DIM)),
            _full_spec((N_DIM, N_DIM)),
            _full_spec((1, N_DIM)),
            _full_spec(((N_DIM + E_DIM) * AXIS, N_DIM)),
            _full_spec((1, N_DIM)),
            _full_spec((1, N_DIM)),
            _full_spec((1, N_DIM)),
            _full_spec((1, N_DIM)),
        ],
        out_specs=[_row_spec(NBLK, (NLOC, N_DIM))],
        out_shape=(jax.ShapeDtypeStruct((NLOC, N_DIM), jnp.float32),),
    )(node_ebd, h2g2_e, h2g2_n, msg, W_ns, b_ns[None, :], W_sym, b_sym[None, :],
      n_res0[None, :], n_res1[None, :], n_res2[None, :])[0]


def _fin_body(ep, red, W, b, eres, out):
    dot = functools.partial(jnp.dot, preferred_element_type=jnp.float32)
    pre = dot(red[...] * (DYN_A ** -0.5), W[...]) + b[...]
    out[...] = ep[...] + eres[...] * _silu(pre)


def _tc_edge_fin(e_part, reduced, W_ea2, b_ea2, e_res1):
    grid = (NEDGE // EBLK,)
    return pl.pallas_call(
        _fin_body,
        grid=grid,
        in_specs=[
            _row_spec(EBLK, (NEDGE, E_DIM)),
            _row_spec(EBLK, (NEDGE, E_DIM)),
            _full_spec((E_DIM, E_DIM)),
            _full_spec((1, E_DIM)),
            _full_spec((1, E_DIM)),
        ],
        out_specs=[_row_spec(EBLK, (NEDGE, E_DIM))],
        out_shape=(jax.ShapeDtypeStruct((NEDGE, E_DIM), jnp.float32),),
    )(e_part, reduced, W_ea2, b_ea2[None, :], e_res1[None, :])[0]


def _gather_rows(table, idx):
    return jnp.take(table, idx, axis=0)


def _segsum(data, owner, num):
    return jax.ops.segment_sum(data, owner, num_segments=num)


def kernel(node_ebd_ext, edge_ebd, h2, angle_ebd, nlist, nlist_mask, sw,
           a_nlist, a_nlist_mask, a_sw, edge_index, angle_index, W_ns, b_ns,
           W_sym, b_sym, W_ne, b_ne, W_es, b_es, W_ea1, b_ea1, W_ea2, b_ea2,
           W_as, b_as, n_res0, n_res1, n_res2, e_res0, e_res1, a_res0):
    node_ext_flat = node_ebd_ext.reshape(-1, N_DIM)
    node_ebd = node_ext_flat[:NLOC]
    n2e = edge_index[0]
    nx2e = edge_index[1]
    n2a = angle_index[0]
    eij2a = angle_index[1]
    eik2a = angle_index[2]

    nei = _gather_rows(node_ext_flat, nx2e)
    node_i = _gather_rows(node_ext_flat, n2e)
    node_a = _gather_rows(node_ext_flat, n2a)
    edge_ik = _gather_rows(edge_ebd, eik2a)
    edge_ij = _gather_rows(edge_ebd, eij2a)

    neu, e_part, w3 = _tc_edge_mlp(node_i, nei, edge_ebd, sw[:, None], h2,
                                   W_ne, b_ne, W_es, b_es, e_res0)
    eaw, a_updated = _tc_angle_mlp(angle_ebd, node_a, edge_ik, edge_ij,
                                   a_sw[:, None], W_ea1, b_ea1, W_as, b_as,
                                   a_res0)

    h2g2_e = jnp.stack([_segsum(w3[:, c:c + 1] * edge_ebd, n2e, NLOC)
                        for c in range(3)], axis=0)
    h2g2_n = jnp.stack([_segsum(w3[:, c:c + 1] * nei, n2e, NLOC)
                        for c in range(3)], axis=0)
    msg = _segsum(neu, n2e, NLOC)
    reduced = _segsum(eaw, eij2a, NEDGE)

    n_updated = _tc_node(node_ebd, h2g2_e, h2g2_n, msg, W_ns, b_ns, W_sym,
                         b_sym, n_res0, n_res1, n_res2)
    e_updated = _tc_edge_fin(e_part, reduced, W_ea2, b_ea2, e_res1)

    return (n_updated.reshape(1, NLOC, N_DIM), e_updated, a_updated)
